# baseline (speedup 1.0000x reference)
import functools
import math

import jax
import jax.numpy as jnp
from jax.experimental import pallas as pl
from jax.experimental.pallas import tpu as pltpu

_S = 2048
_HQ = 16
_HKV = 4
_REP = _HQ // _HKV
_D = 128
_KER = 32
_STR = 16
_BLK = 64
_TOPN = 16
_WIN = 512
_M = (_S - _KER) // _STR + 1
_NB = _S // _BLK
_NQ = _S // _BLK
_NEG = -1e30


def _nsa_kernel(qt_ref, ktb_ref, vtb_ref, ka_ref, kb_ref, va_ref, vb_ref,
                b1k_ref, b2k_ref, b1v_ref, b2v_ref, ws_ref, gw_ref,
                out_ref, cks, cvs, m2_s, l2_s, a2_s, m3_s, l3_s, a3_s):
    qi = pl.program_id(1)
    scale = 1.0 / math.sqrt(_D)

    @pl.when(qi == 0)
    def _compress_kv():
        wsk = ws_ref[0:1, 0:1]
        wsv = ws_ref[1:2, 0:1]
        ck = (jnp.dot(b1k_ref[...], ka_ref[0], preferred_element_type=jnp.float32)
              + jnp.dot(b2k_ref[...], kb_ref[0], preferred_element_type=jnp.float32))
        cv = (jnp.dot(b1v_ref[...], va_ref[0], preferred_element_type=jnp.float32)
              + jnp.dot(b2v_ref[...], vb_ref[0], preferred_element_type=jnp.float32))
        cks[...] = (ck / wsk).astype(jnp.bfloat16)
        cvs[...] = (cv / wsv).astype(jnp.bfloat16)

    q3 = qt_ref[0]
    q2 = q3.reshape(_REP * _BLK, _D).astype(jnp.bfloat16)

    t_row = _BLK * qi + jax.lax.broadcasted_iota(jnp.int32, (1, _BLK, 1), 1)

    sc = jax.lax.dot_general(q2, cks[...], (((1,), (1,)), ((), ())),
                             preferred_element_type=jnp.float32) * scale
    sc3 = sc.reshape(_REP, _BLK, 128)
    m_col = jax.lax.broadcasted_iota(jnp.int32, (1, 1, 128), 2)
    cmask = (_STR * m_col + _KER - 1 <= t_row) & (m_col < _M)
    scm = jnp.where(cmask, sc3, _NEG)
    cmx = jnp.max(scm, axis=2, keepdims=True)
    ce = jnp.where(cmask, jnp.exp(scm - cmx), 0.0)
    cden = jnp.sum(ce, axis=2, keepdims=True)
    pc = ce / jnp.maximum(cden, 1e-20)
    out_cmp = jnp.dot(pc.reshape(_REP * _BLK, 128).astype(jnp.bfloat16),
                      cvs[...], preferred_element_type=jnp.float32)

    pg = jnp.sum(pc, axis=0)
    sel = jnp.sum(pg.reshape(_BLK, _NB, 4), axis=2)
    nn = jax.lax.broadcasted_iota(jnp.int32, (_BLK, _NB), 1)
    force = (nn == qi) | (nn == 0)
    sel = sel + jnp.where(force, 1e9, 0.0)
    selw = sel
    picked = jnp.zeros((_BLK, _NB), jnp.bool_)
    for _ in range(_TOPN):
        mx = jnp.max(selw, axis=1, keepdims=True)
        idx = jnp.where(selw == mx, nn, _NB)
        fidx = jnp.min(idx, axis=1, keepdims=True)
        pick = nn == fidx
        picked = picked | pick
        selw = jnp.where(pick, -jnp.inf, selw)
    blk = picked.astype(jnp.float32)

    m2_s[...] = jnp.full((_REP, _BLK, 1), _NEG, jnp.float32)
    l2_s[...] = jnp.zeros((_REP, _BLK, 1), jnp.float32)
    a2_s[...] = jnp.zeros((_REP * _BLK, _D), jnp.float32)
    m3_s[...] = jnp.full((_REP, _BLK, 1), _NEG, jnp.float32)
    l3_s[...] = jnp.zeros((_REP, _BLK, 1), jnp.float32)
    a3_s[...] = jnp.zeros((_REP * _BLK, _D), jnp.float32)

    jj = jax.lax.broadcasted_iota(jnp.int32, (1, 1, _BLK), 2)

    def upd(mask, s3, vs, m_ref, l_ref, a_ref):
        sm = jnp.where(mask, s3, _NEG)
        m_o = m_ref[...]
        m_n = jnp.maximum(m_o, jnp.max(sm, axis=2, keepdims=True))
        alpha = jnp.exp(m_o - m_n)
        e = jnp.where(mask, jnp.exp(sm - m_n), 0.0)
        l_ref[...] = l_ref[...] * alpha + jnp.sum(e, axis=2, keepdims=True)
        a_ref[...] = a_ref[...] * alpha.reshape(_REP * _BLK, 1) + jnp.dot(
            e.reshape(_REP * _BLK, _BLK).astype(jnp.bfloat16), vs,
            preferred_element_type=jnp.float32)
        m_ref[...] = m_n

    for n in range(_NQ):
        @pl.when(n <= qi)
        def _step(n=n):
            ks = ktb_ref[0, n * _BLK:(n + 1) * _BLK, :]
            vs = vtb_ref[0, n * _BLK:(n + 1) * _BLK, :]
            s = jax.lax.dot_general(q2, ks, (((1,), (1,)), ((), ())),
                                    preferred_element_type=jnp.float32) * scale
            s3 = s.reshape(_REP, _BLK, _BLK)
            s_tok = _BLK * n + jj
            causal = s_tok <= t_row
            bcol = blk[:, n:n + 1]
            mask2 = causal & (bcol[None] > 0.5)
            upd(mask2, s3, vs, m2_s, l2_s, a2_s)

            @pl.when(n >= qi - (_WIN // _BLK))
            def _win():
                mask3 = causal & (s_tok > t_row - _WIN)
                upd(mask3, s3, vs, m3_s, l3_s, a3_s)

    out_sel = a2_s[...] / jnp.maximum(l2_s[...].reshape(_REP * _BLK, 1), 1e-20)
    out_win = a3_s[...] / jnp.maximum(l3_s[...].reshape(_REP * _BLK, 1), 1e-20)

    g = jax.nn.sigmoid(jnp.dot(q2, gw_ref[...],
                               preferred_element_type=jnp.float32))
    out = (g[:, 0:1] * out_cmp + g[:, 1:2] * out_sel + g[:, 2:3] * out_win)
    out_ref[0] = out.reshape(_REP, _BLK, _D)


def _half_band(w_half, lo):
    off = jnp.arange(_S)[None, :] - _STR * jnp.arange(_M)[:, None] - lo
    valid = (off >= 0) & (off < _STR)
    band = jnp.where(valid, w_half[jnp.clip(off, 0, _STR - 1)], 0.0)
    return jnp.pad(band, ((0, 1), (0, 0))).astype(jnp.bfloat16)


@functools.partial(jax.jit, static_argnames=("interpret",))
def _nsa(q, k, v, w_k, w_v, pe_k, pe_v, gate_w, interpret=False):
    qt = q[0].reshape(_S, _HKV, _REP, _D).transpose(1, 2, 0, 3)
    kt = k[0].transpose(1, 0, 2)
    vt = v[0].transpose(1, 0, 2)
    ktb = kt.astype(jnp.bfloat16)
    vtb = vt.astype(jnp.bfloat16)

    pea_k = jnp.tile(pe_k[:_STR], (_S // _STR, 1))
    peb_k = jnp.tile(pe_k[_STR:], (_S // _STR, 1))
    pea_v = jnp.tile(pe_v[:_STR], (_S // _STR, 1))
    peb_v = jnp.tile(pe_v[_STR:], (_S // _STR, 1))
    ka = (kt + pea_k[None]).astype(jnp.bfloat16)
    kb = (kt + peb_k[None]).astype(jnp.bfloat16)
    va = (vt + pea_v[None]).astype(jnp.bfloat16)
    vb = (vt + peb_v[None]).astype(jnp.bfloat16)

    b1k = _half_band(w_k[:_STR], 0)
    b2k = _half_band(w_k[_STR:], _STR)
    b1v = _half_band(w_v[:_STR], 0)
    b2v = _half_band(w_v[_STR:], _STR)

    ws = jnp.zeros((8, 128), jnp.float32)
    ws = ws.at[0, 0].set(jnp.maximum(jnp.sum(w_k), 1e-6))
    ws = ws.at[1, 0].set(jnp.maximum(jnp.sum(w_v), 1e-6))
    gw = jnp.pad(gate_w, ((0, 0), (0, 5))).astype(jnp.bfloat16)

    out_t = pl.pallas_call(
        _nsa_kernel,
        grid=(_HKV, _NQ),
        in_specs=[
            pl.BlockSpec((1, _REP, _BLK, _D), lambda h, qi: (h, 0, qi, 0)),
            pl.BlockSpec((1, _S, _D), lambda h, qi: (h, 0, 0)),
            pl.BlockSpec((1, _S, _D), lambda h, qi: (h, 0, 0)),
            pl.BlockSpec((1, _S, _D), lambda h, qi: (h, 0, 0)),
            pl.BlockSpec((1, _S, _D), lambda h, qi: (h, 0, 0)),
            pl.BlockSpec((1, _S, _D), lambda h, qi: (h, 0, 0)),
            pl.BlockSpec((1, _S, _D), lambda h, qi: (h, 0, 0)),
            pl.BlockSpec((128, _S), lambda h, qi: (0, 0)),
            pl.BlockSpec((128, _S), lambda h, qi: (0, 0)),
            pl.BlockSpec((128, _S), lambda h, qi: (0, 0)),
            pl.BlockSpec((128, _S), lambda h, qi: (0, 0)),
            pl.BlockSpec((8, 128), lambda h, qi: (0, 0)),
            pl.BlockSpec((_D, 8), lambda h, qi: (0, 0)),
        ],
        out_specs=pl.BlockSpec((1, _REP, _BLK, _D), lambda h, qi: (h, 0, qi, 0)),
        out_shape=jax.ShapeDtypeStruct((_HKV, _REP, _S, _D), jnp.float32),
        scratch_shapes=[
            pltpu.VMEM((128, _D), jnp.bfloat16),
            pltpu.VMEM((128, _D), jnp.bfloat16),
            pltpu.VMEM((_REP, _BLK, 1), jnp.float32),
            pltpu.VMEM((_REP, _BLK, 1), jnp.float32),
            pltpu.VMEM((_REP * _BLK, _D), jnp.float32),
            pltpu.VMEM((_REP, _BLK, 1), jnp.float32),
            pltpu.VMEM((_REP, _BLK, 1), jnp.float32),
            pltpu.VMEM((_REP * _BLK, _D), jnp.float32),
        ],
        interpret=interpret,
    )(qt, ktb, vtb, ka, kb, va, vb, b1k, b2k, b1v, b2v, ws, gw)

    out = out_t.transpose(2, 0, 1, 3).reshape(1, _S, _HQ, _D)
    return out


def kernel(q, k, v, w_k, w_v, pe_k, pe_v, gate_w):
    return _nsa(q, k, v, w_k, w_v, pe_k, pe_v, gate_w)


# speedup vs baseline: 1.5783x; 1.5783x over previous
import functools
import math

import jax
import jax.numpy as jnp
from jax.experimental import pallas as pl
from jax.experimental.pallas import tpu as pltpu

_S = 2048
_HQ = 16
_HKV = 4
_REP = _HQ // _HKV
_D = 128
_KER = 32
_STR = 16
_BLK = 64
_TOPN = 16
_WIN = 512
_M = (_S - _KER) // _STR + 1
_NB = _S // _BLK
_NQ = _S // _BLK
_CH = 256
_NEG = -1e30


def _nsa_kernel(qt_ref, ktb_ref, vtb_ref, ka_ref, kb_ref, va_ref, vb_ref,
                b1k_ref, b2k_ref, b1v_ref, b2v_ref, ws_ref, gw_ref, e_ref,
                out_ref, cks, cvs, m2_s, l2_s, a2_s, m3_s, l3_s, a3_s):
    qi = pl.program_id(1)
    scale = 1.0 / math.sqrt(_D)

    @pl.when(qi == 0)
    def _compress_kv():
        wsk = ws_ref[0:1, 0:1]
        wsv = ws_ref[1:2, 0:1]
        ck = (jnp.dot(b1k_ref[...], ka_ref[0], preferred_element_type=jnp.float32)
              + jnp.dot(b2k_ref[...], kb_ref[0], preferred_element_type=jnp.float32))
        cv = (jnp.dot(b1v_ref[...], va_ref[0], preferred_element_type=jnp.float32)
              + jnp.dot(b2v_ref[...], vb_ref[0], preferred_element_type=jnp.float32))
        cks[...] = (ck / wsk).astype(jnp.bfloat16)
        cvs[...] = (cv / wsv).astype(jnp.bfloat16)

    q3 = qt_ref[0]
    q2 = q3.reshape(_REP * _BLK, _D).astype(jnp.bfloat16)

    t_row = _BLK * qi + jax.lax.broadcasted_iota(jnp.int32, (1, _BLK, 1), 1)

    sc = jax.lax.dot_general(q2, cks[...], (((1,), (1,)), ((), ())),
                             preferred_element_type=jnp.float32) * scale
    sc3 = sc.reshape(_REP, _BLK, 128)
    m_col = jax.lax.broadcasted_iota(jnp.int32, (1, 1, 128), 2)
    cmask = (_STR * m_col + _KER - 1 <= t_row) & (m_col < _M)
    scm = jnp.where(cmask, sc3, _NEG)
    cmx = jnp.max(scm, axis=2, keepdims=True)
    ce = jnp.where(cmask, jnp.exp(scm - cmx), 0.0)
    cden = jnp.sum(ce, axis=2, keepdims=True)
    pc = ce / jnp.maximum(cden, 1e-20)
    out_cmp = jnp.dot(pc.reshape(_REP * _BLK, 128).astype(jnp.bfloat16),
                      cvs[...], preferred_element_type=jnp.float32)

    pg = jnp.sum(pc, axis=0)
    sel = jnp.sum(pg.reshape(_BLK, _NB, 4), axis=2)
    nn = jax.lax.broadcasted_iota(jnp.int32, (_BLK, _NB), 1)
    force = (nn == qi) | (nn == 0)
    sel = sel + jnp.where(force, 1e9, 0.0)
    selw = sel
    picked = jnp.zeros((_BLK, _NB), jnp.bool_)
    for _ in range(_TOPN):
        mx = jnp.max(selw, axis=1, keepdims=True)
        idx = jnp.where(selw == mx, nn, _NB)
        fidx = jnp.min(idx, axis=1, keepdims=True)
        pick = nn == fidx
        picked = picked | pick
        selw = jnp.where(pick, -jnp.inf, selw)
    blk = picked.astype(jnp.float32)

    blk_tok = jnp.dot(blk.astype(jnp.bfloat16), e_ref[...],
                      preferred_element_type=jnp.float32)

    m2_s[...] = jnp.full((_REP, _BLK, 1), _NEG, jnp.float32)
    l2_s[...] = jnp.zeros((_REP, _BLK, 1), jnp.float32)
    a2_s[...] = jnp.zeros((_REP * _BLK, _D), jnp.float32)
    m3_s[...] = jnp.full((_REP, _BLK, 1), _NEG, jnp.float32)
    l3_s[...] = jnp.zeros((_REP, _BLK, 1), jnp.float32)
    a3_s[...] = jnp.zeros((_REP * _BLK, _D), jnp.float32)

    jj = jax.lax.broadcasted_iota(jnp.int32, (1, 1, _CH), 2)

    def upd(mask, s3, vs, m_ref, l_ref, a_ref):
        sm = jnp.where(mask, s3, _NEG)
        m_o = m_ref[...]
        m_n = jnp.maximum(m_o, jnp.max(sm, axis=2, keepdims=True))
        alpha = jnp.exp(m_o - m_n)
        e = jnp.where(mask, jnp.exp(sm - m_n), 0.0)
        l_ref[...] = l_ref[...] * alpha + jnp.sum(e, axis=2, keepdims=True)
        a_ref[...] = a_ref[...] * alpha.reshape(_REP * _BLK, 1) + jnp.dot(
            e.reshape(_REP * _BLK, _CH).astype(jnp.bfloat16), vs,
            preferred_element_type=jnp.float32)
        m_ref[...] = m_n

    for c in range(_S // _CH):
        @pl.when(4 * c <= qi)
        def _step(c=c):
            ks = ktb_ref[0, c * _CH:(c + 1) * _CH, :]
            vs = vtb_ref[0, c * _CH:(c + 1) * _CH, :]
            s = jax.lax.dot_general(q2, ks, (((1,), (1,)), ((), ())),
                                    preferred_element_type=jnp.float32) * scale
            s3 = s.reshape(_REP, _BLK, _CH)
            s_tok = _CH * c + jj
            causal = s_tok <= t_row
            bcol = blk_tok[:, c * _CH:(c + 1) * _CH]
            mask2 = causal & (bcol[None] > 0.5)
            upd(mask2, s3, vs, m2_s, l2_s, a2_s)

            @pl.when(4 * c >= qi - 11)
            def _win():
                mask3 = causal & (s_tok > t_row - _WIN)
                upd(mask3, s3, vs, m3_s, l3_s, a3_s)

    out_sel = a2_s[...] / jnp.maximum(l2_s[...].reshape(_REP * _BLK, 1), 1e-20)
    out_win = a3_s[...] / jnp.maximum(l3_s[...].reshape(_REP * _BLK, 1), 1e-20)

    g = jax.nn.sigmoid(jnp.dot(q2, gw_ref[...],
                               preferred_element_type=jnp.float32))
    out = (g[:, 0:1] * out_cmp + g[:, 1:2] * out_sel + g[:, 2:3] * out_win)
    out_ref[0] = out.reshape(_REP, _BLK, _D)


def _half_band(w_half, lo):
    off = jnp.arange(_S)[None, :] - _STR * jnp.arange(_M)[:, None] - lo
    valid = (off >= 0) & (off < _STR)
    band = jnp.where(valid, w_half[jnp.clip(off, 0, _STR - 1)], 0.0)
    return jnp.pad(band, ((0, 1), (0, 0))).astype(jnp.bfloat16)


@functools.partial(jax.jit, static_argnames=("interpret",))
def _nsa(q, k, v, w_k, w_v, pe_k, pe_v, gate_w, interpret=False):
    qt = q[0].reshape(_S, _HKV, _REP, _D).transpose(1, 2, 0, 3)
    kt = k[0].transpose(1, 0, 2)
    vt = v[0].transpose(1, 0, 2)
    ktb = kt.astype(jnp.bfloat16)
    vtb = vt.astype(jnp.bfloat16)

    pea_k = jnp.tile(pe_k[:_STR], (_S // _STR, 1))
    peb_k = jnp.tile(pe_k[_STR:], (_S // _STR, 1))
    pea_v = jnp.tile(pe_v[:_STR], (_S // _STR, 1))
    peb_v = jnp.tile(pe_v[_STR:], (_S // _STR, 1))
    ka = (kt + pea_k[None]).astype(jnp.bfloat16)
    kb = (kt + peb_k[None]).astype(jnp.bfloat16)
    va = (vt + pea_v[None]).astype(jnp.bfloat16)
    vb = (vt + peb_v[None]).astype(jnp.bfloat16)

    b1k = _half_band(w_k[:_STR], 0)
    b2k = _half_band(w_k[_STR:], _STR)
    b1v = _half_band(w_v[:_STR], 0)
    b2v = _half_band(w_v[_STR:], _STR)

    ws = jnp.zeros((8, 128), jnp.float32)
    ws = ws.at[0, 0].set(jnp.maximum(jnp.sum(w_k), 1e-6))
    ws = ws.at[1, 0].set(jnp.maximum(jnp.sum(w_v), 1e-6))
    gw = jnp.pad(gate_w, ((0, 0), (0, 5))).astype(jnp.bfloat16)
    emat = (jnp.arange(_S)[None, :] // _BLK
            == jnp.arange(_NB)[:, None]).astype(jnp.bfloat16)

    out_t = pl.pallas_call(
        _nsa_kernel,
        grid=(_HKV, _NQ),
        in_specs=[
            pl.BlockSpec((1, _REP, _BLK, _D), lambda h, qi: (h, 0, qi, 0)),
            pl.BlockSpec((1, _S, _D), lambda h, qi: (h, 0, 0)),
            pl.BlockSpec((1, _S, _D), lambda h, qi: (h, 0, 0)),
            pl.BlockSpec((1, _S, _D), lambda h, qi: (h, 0, 0)),
            pl.BlockSpec((1, _S, _D), lambda h, qi: (h, 0, 0)),
            pl.BlockSpec((1, _S, _D), lambda h, qi: (h, 0, 0)),
            pl.BlockSpec((1, _S, _D), lambda h, qi: (h, 0, 0)),
            pl.BlockSpec((128, _S), lambda h, qi: (0, 0)),
            pl.BlockSpec((128, _S), lambda h, qi: (0, 0)),
            pl.BlockSpec((128, _S), lambda h, qi: (0, 0)),
            pl.BlockSpec((128, _S), lambda h, qi: (0, 0)),
            pl.BlockSpec((8, 128), lambda h, qi: (0, 0)),
            pl.BlockSpec((_D, 8), lambda h, qi: (0, 0)),
            pl.BlockSpec((_NB, _S), lambda h, qi: (0, 0)),
        ],
        out_specs=pl.BlockSpec((1, _REP, _BLK, _D), lambda h, qi: (h, 0, qi, 0)),
        out_shape=jax.ShapeDtypeStruct((_HKV, _REP, _S, _D), jnp.float32),
        scratch_shapes=[
            pltpu.VMEM((128, _D), jnp.bfloat16),
            pltpu.VMEM((128, _D), jnp.bfloat16),
            pltpu.VMEM((_REP, _BLK, 1), jnp.float32),
            pltpu.VMEM((_REP, _BLK, 1), jnp.float32),
            pltpu.VMEM((_REP * _BLK, _D), jnp.float32),
            pltpu.VMEM((_REP, _BLK, 1), jnp.float32),
            pltpu.VMEM((_REP, _BLK, 1), jnp.float32),
            pltpu.VMEM((_REP * _BLK, _D), jnp.float32),
        ],
        interpret=interpret,
    )(qt, ktb, vtb, ka, kb, va, vb, b1k, b2k, b1v, b2v, ws, gw, emat)

    out = out_t.transpose(2, 0, 1, 3).reshape(1, _S, _HQ, _D)
    return out


def kernel(q, k, v, w_k, w_v, pe_k, pe_v, gate_w):
    return _nsa(q, k, v, w_k, w_v, pe_k, pe_v, gate_w)


# speedup vs baseline: 2.9788x; 1.8873x over previous
import functools
import math

import jax
import jax.numpy as jnp
from jax.experimental import pallas as pl
from jax.experimental.pallas import tpu as pltpu

_S = 2048
_HQ = 16
_HKV = 4
_REP = _HQ // _HKV
_D = 128
_KER = 32
_STR = 16
_BLK = 64
_TOPN = 16
_WIN = 512
_M = (_S - _KER) // _STR + 1
_NB = _S // _BLK
_NQ = _S // _BLK
_CH = 256
_R = _REP * _BLK
_NEG = -1e30
_HI = jax.lax.Precision.HIGHEST


def _nsa_kernel(qt_ref, ktb_ref, vtb_ref, ka_ref, kb_ref, va_ref, vb_ref,
                b1k_ref, b2k_ref, b1v_ref, b2v_ref, ws_ref, gw_ref,
                etok_ref, rsum_ref, em_ref, out_ref, cks, cvs, w2_s):
    qi = pl.program_id(1)
    scale = 1.0 / math.sqrt(_D)

    @pl.when(qi == 0)
    def _compress_kv():
        wsk = ws_ref[0:1, 0:1]
        wsv = ws_ref[1:2, 0:1]
        ck = (jnp.dot(b1k_ref[...], ka_ref[0], preferred_element_type=jnp.float32)
              + jnp.dot(b2k_ref[...], kb_ref[0], preferred_element_type=jnp.float32))
        cv = (jnp.dot(b1v_ref[...], va_ref[0], preferred_element_type=jnp.float32)
              + jnp.dot(b2v_ref[...], vb_ref[0], preferred_element_type=jnp.float32))
        cks[...] = (ck / wsk).astype(jnp.bfloat16)
        cvs[...] = (cv / wsv).astype(jnp.bfloat16)

    q2 = qt_ref[0].reshape(_R, _D).astype(jnp.bfloat16)
    tq = jax.lax.broadcasted_iota(jnp.int32, (1, _R), 1) % _BLK
    tval = _BLK * qi + tq

    scT = jax.lax.dot_general(cks[...], q2, (((1,), (1,)), ((), ())),
                              preferred_element_type=jnp.float32)
    m_sub = jax.lax.broadcasted_iota(jnp.int32, (128, 1), 0)
    cadd = jnp.where((_STR * m_sub + _KER - 1 <= tval) & (m_sub < _M),
                     0.0, _NEG)
    scm = scT * scale + cadd
    cmx = jnp.max(scm, axis=0, keepdims=True)
    ce = jnp.exp(scm - cmx)
    cden = jnp.sum(ce, axis=0, keepdims=True)
    pcT = ce / jnp.maximum(cden, 1e-20)
    out_cmpT = jax.lax.dot_general(cvs[...], pcT.astype(jnp.bfloat16),
                                   (((0,), (0,)), ((), ())),
                                   preferred_element_type=jnp.float32)
    out_cmpT = out_cmpT * jnp.where(tval >= _KER - 1, 1.0, 0.0)

    pgT = jax.lax.dot_general(pcT, rsum_ref[...], (((1,), (0,)), ((), ())),
                              precision=_HI,
                              preferred_element_type=jnp.float32)
    selT = jax.lax.dot_general(em_ref[...], pgT, (((1,), (0,)), ((), ())),
                               precision=_HI,
                               preferred_element_type=jnp.float32)
    nnS = jax.lax.broadcasted_iota(jnp.int32, (_NB, 1), 0)
    selT = selT + jnp.where((nnS == qi) | (nnS == 0), 1e9, 0.0)
    selw = selT
    picked = jnp.zeros((_NB, _BLK), jnp.bool_)
    for _ in range(_TOPN):
        mx = jnp.max(selw, axis=0, keepdims=True)
        idx = jnp.where(selw == mx, nnS, _NB)
        fidx = jnp.min(idx, axis=0, keepdims=True)
        pick = nnS == fidx
        picked = picked | pick
        selw = jnp.where(pick, -jnp.inf, selw)
    blk_add = jnp.where(picked, 0.0, _NEG).astype(jnp.bfloat16)
    blk_add4 = jnp.concatenate([blk_add] * _REP, axis=1)
    w2_s[...] = jax.lax.dot_general(etok_ref[...], blk_add4,
                                    (((1,), (0,)), ((), ())),
                                    preferred_element_type=jnp.float32)

    cq = qi // 4

    def qk(c):
        ks = ktb_ref[0, pl.ds(c * _CH, _CH), :]
        vs = vtb_ref[0, pl.ds(c * _CH, _CH), :]
        sT = jax.lax.dot_general(ks, q2, (((1,), (1,)), ((), ())),
                                 preferred_element_type=jnp.float32)
        return sT, vs

    def upd(sm, vs, carry):
        m_o, l_o, acc = carry
        m_n = jnp.maximum(m_o, jnp.max(sm, axis=0, keepdims=True))
        alpha = jnp.exp(m_o - m_n)
        e = jnp.exp(sm - m_n)
        l_n = l_o * alpha + jnp.sum(e, axis=0, keepdims=True)
        pv = jax.lax.dot_general(vs, e.astype(jnp.bfloat16),
                                 (((0,), (0,)), ((), ())),
                                 preferred_element_type=jnp.float32)
        return m_n, l_n, acc * alpha + pv

    init = (jnp.full((1, _R), _NEG, jnp.float32),
            jnp.zeros((1, _R), jnp.float32),
            jnp.zeros((_D, _R), jnp.float32))

    def body2(c, carry):
        sT, vs = qk(c)
        sm = sT * scale + w2_s[pl.ds(c * _CH, _CH), :]
        return upd(sm, vs, carry)

    car2 = jax.lax.fori_loop(0, cq, body2, init)

    jsub = jax.lax.broadcasted_iota(jnp.int32, (_CH, 1), 0)

    def body3(c, carry):
        sT, vs = qk(c)
        sm = sT * scale + jnp.where(_CH * c + jsub > tval - _WIN, 0.0, _NEG)
        return upd(sm, vs, carry)

    car3 = jax.lax.fori_loop(jnp.maximum(cq - 2, 0), cq, body3, init)

    sT, vs = qk(cq)
    ssc = sT * scale
    cadd2 = jnp.where(_CH * cq + jsub <= tval, 0.0, _NEG)
    car2 = upd(ssc + w2_s[pl.ds(cq * _CH, _CH), :] + cadd2, vs, car2)
    car3 = upd(ssc + cadd2, vs, car3)

    out_selT = car2[2] / jnp.maximum(car2[1], 1e-20)
    out_winT = car3[2] / jnp.maximum(car3[1], 1e-20)

    gT = jax.nn.sigmoid(jax.lax.dot_general(
        gw_ref[...], q2, (((0,), (1,)), ((), ())),
        preferred_element_type=jnp.float32))
    outT = (gT[0:1] * out_cmpT + gT[1:2] * out_selT + gT[2:3] * out_winT)
    out_ref[0, 0] = outT


def _half_band(w_half, lo):
    off = jnp.arange(_S)[None, :] - _STR * jnp.arange(_M)[:, None] - lo
    valid = (off >= 0) & (off < _STR)
    band = jnp.where(valid, w_half[jnp.clip(off, 0, _STR - 1)], 0.0)
    return jnp.pad(band, ((0, 1), (0, 0))).astype(jnp.bfloat16)


@functools.partial(jax.jit, static_argnames=("interpret",))
def _nsa(q, k, v, w_k, w_v, pe_k, pe_v, gate_w, interpret=False):
    qt = q[0].reshape(_S, _HKV, _REP, _D).transpose(1, 2, 0, 3)
    kt = k[0].transpose(1, 0, 2)
    vt = v[0].transpose(1, 0, 2)
    ktb = kt.astype(jnp.bfloat16)
    vtb = vt.astype(jnp.bfloat16)

    pea_k = jnp.tile(pe_k[:_STR], (_S // _STR, 1))
    peb_k = jnp.tile(pe_k[_STR:], (_S // _STR, 1))
    pea_v = jnp.tile(pe_v[:_STR], (_S // _STR, 1))
    peb_v = jnp.tile(pe_v[_STR:], (_S // _STR, 1))
    ka = (kt + pea_k[None]).astype(jnp.bfloat16)
    kb = (kt + peb_k[None]).astype(jnp.bfloat16)
    va = (vt + pea_v[None]).astype(jnp.bfloat16)
    vb = (vt + peb_v[None]).astype(jnp.bfloat16)

    b1k = _half_band(w_k[:_STR], 0)
    b2k = _half_band(w_k[_STR:], _STR)
    b1v = _half_band(w_v[:_STR], 0)
    b2v = _half_band(w_v[_STR:], _STR)

    ws = jnp.zeros((8, 128), jnp.float32)
    ws = ws.at[0, 0].set(jnp.maximum(jnp.sum(w_k), 1e-6))
    ws = ws.at[1, 0].set(jnp.maximum(jnp.sum(w_v), 1e-6))
    gw = jnp.pad(gate_w, ((0, 0), (0, 5))).astype(jnp.bfloat16)

    etok = (jnp.arange(_S)[:, None] // _BLK
            == jnp.arange(_NB)[None, :]).astype(jnp.bfloat16)
    rsum = (jnp.arange(_R)[:, None] % _BLK
            == jnp.arange(_BLK)[None, :]).astype(jnp.float32)
    em = (jnp.arange(128)[None, :] // 4
          == jnp.arange(_NB)[:, None]).astype(jnp.float32)

    out_t = pl.pallas_call(
        _nsa_kernel,
        grid=(_HKV, _NQ),
        in_specs=[
            pl.BlockSpec((1, _REP, _BLK, _D), lambda h, qi: (h, 0, qi, 0)),
            pl.BlockSpec((1, _S, _D), lambda h, qi: (h, 0, 0)),
            pl.BlockSpec((1, _S, _D), lambda h, qi: (h, 0, 0)),
            pl.BlockSpec((1, _S, _D), lambda h, qi: (h, 0, 0)),
            pl.BlockSpec((1, _S, _D), lambda h, qi: (h, 0, 0)),
            pl.BlockSpec((1, _S, _D), lambda h, qi: (h, 0, 0)),
            pl.BlockSpec((1, _S, _D), lambda h, qi: (h, 0, 0)),
            pl.BlockSpec((128, _S), lambda h, qi: (0, 0)),
            pl.BlockSpec((128, _S), lambda h, qi: (0, 0)),
            pl.BlockSpec((128, _S), lambda h, qi: (0, 0)),
            pl.BlockSpec((128, _S), lambda h, qi: (0, 0)),
            pl.BlockSpec((8, 128), lambda h, qi: (0, 0)),
            pl.BlockSpec((_D, 8), lambda h, qi: (0, 0)),
            pl.BlockSpec((_S, _NB), lambda h, qi: (0, 0)),
            pl.BlockSpec((_R, _BLK), lambda h, qi: (0, 0)),
            pl.BlockSpec((_NB, 128), lambda h, qi: (0, 0)),
        ],
        out_specs=pl.BlockSpec((1, 1, _D, _R), lambda h, qi: (h, qi, 0, 0)),
        out_shape=jax.ShapeDtypeStruct((_HKV, _NQ, _D, _R), jnp.float32),
        scratch_shapes=[
            pltpu.VMEM((128, _D), jnp.bfloat16),
            pltpu.VMEM((128, _D), jnp.bfloat16),
            pltpu.VMEM((_S, _R), jnp.float32),
        ],
        interpret=interpret,
    )(qt, ktb, vtb, ka, kb, va, vb, b1k, b2k, b1v, b2v, ws, gw,
      etok, rsum, em)

    out = (out_t.reshape(_HKV, _NQ, _D, _REP, _BLK)
           .transpose(1, 4, 0, 3, 2).reshape(1, _S, _HQ, _D))
    return out


def kernel(q, k, v, w_k, w_v, pe_k, pe_v, gate_w):
    return _nsa(q, k, v, w_k, w_v, pe_k, pe_v, gate_w)


# speedup vs baseline: 5.2737x; 1.7704x over previous
import functools
import math

import jax
import jax.numpy as jnp
from jax.experimental import pallas as pl
from jax.experimental.pallas import tpu as pltpu

_S = 2048
_HQ = 16
_HKV = 4
_REP = _HQ // _HKV
_D = 128
_KER = 32
_STR = 16
_BLK = 64
_TOPN = 16
_WIN = 512
_M = (_S - _KER) // _STR + 1
_NB = _S // _BLK
_G = 256
_NG = _S // _G
_CH = 256
_R = _REP * _G
_NEG = -1e30
_HI = jax.lax.Precision.HIGHEST


def _nsa_kernel(qt_ref, ktb_ref, vtb_ref, ka_ref, kb_ref, va_ref, vb_ref,
                b1k_ref, b2k_ref, b1v_ref, b2v_ref, ws_ref, gw_ref,
                etok_ref, rsum_ref, em_ref, out_ref, cks, cvs, w2_s):
    qj = pl.program_id(1)
    scale = 1.0 / math.sqrt(_D)

    @pl.when(qj == 0)
    def _compress_kv():
        wsk = ws_ref[0:1, 0:1]
        wsv = ws_ref[1:2, 0:1]
        ck = (jnp.dot(b1k_ref[...], ka_ref[0], preferred_element_type=jnp.float32)
              + jnp.dot(b2k_ref[...], kb_ref[0], preferred_element_type=jnp.float32))
        cv = (jnp.dot(b1v_ref[...], va_ref[0], preferred_element_type=jnp.float32)
              + jnp.dot(b2v_ref[...], vb_ref[0], preferred_element_type=jnp.float32))
        cks[...] = (ck / wsk).astype(jnp.bfloat16)
        cvs[...] = (cv / wsv).astype(jnp.bfloat16)

    q2 = qt_ref[0].reshape(_R, _D).astype(jnp.bfloat16)
    tl = jax.lax.broadcasted_iota(jnp.int32, (1, _R), 1) % _G
    tval = _G * qj + tl

    scT = jax.lax.dot_general(cks[...], q2, (((1,), (1,)), ((), ())),
                              preferred_element_type=jnp.float32)
    m_sub = jax.lax.broadcasted_iota(jnp.int32, (128, 1), 0)
    cadd = jnp.where((_STR * m_sub + _KER - 1 <= tval) & (m_sub < _M),
                     0.0, _NEG)
    scm = scT * scale + cadd
    cmx = jnp.max(scm, axis=0, keepdims=True)
    ce = jnp.exp(scm - cmx)
    cden = jnp.sum(ce, axis=0, keepdims=True)
    pcT = ce / jnp.maximum(cden, 1e-20)
    out_cmpT = jax.lax.dot_general(cvs[...], pcT.astype(jnp.bfloat16),
                                   (((0,), (0,)), ((), ())),
                                   preferred_element_type=jnp.float32)
    out_cmpT = out_cmpT * jnp.where(tval >= _KER - 1, 1.0, 0.0)

    pgT = jax.lax.dot_general(pcT, rsum_ref[...], (((1,), (0,)), ((), ())),
                              precision=_HI,
                              preferred_element_type=jnp.float32)
    selT = jax.lax.dot_general(em_ref[...], pgT, (((1,), (0,)), ((), ())),
                               precision=_HI,
                               preferred_element_type=jnp.float32)
    nnS = jax.lax.broadcasted_iota(jnp.int32, (_NB, 1), 0)
    cur = 4 * qj + jax.lax.broadcasted_iota(jnp.int32, (1, _G), 1) // _BLK
    selT = selT + jnp.where((nnS == cur) | (nnS == 0), 1e9, 0.0)
    selw = selT
    picked = jnp.zeros((_NB, _G), jnp.bool_)
    for _ in range(_TOPN):
        mx = jnp.max(selw, axis=0, keepdims=True)
        idx = jnp.where(selw == mx, nnS, _NB)
        fidx = jnp.min(idx, axis=0, keepdims=True)
        pick = nnS == fidx
        picked = picked | pick
        selw = jnp.where(pick, -jnp.inf, selw)
    blk_add = jnp.where(picked, 0.0, _NEG).astype(jnp.bfloat16)
    blk_add4 = jnp.concatenate([blk_add] * _REP, axis=1)
    w2_s[...] = jax.lax.dot_general(etok_ref[...], blk_add4,
                                    (((1,), (0,)), ((), ())),
                                    preferred_element_type=jnp.float32)

    def qk(c):
        ks = ktb_ref[0, pl.ds(c * _CH, _CH), :]
        vs = vtb_ref[0, pl.ds(c * _CH, _CH), :]
        sT = jax.lax.dot_general(ks, q2, (((1,), (1,)), ((), ())),
                                 preferred_element_type=jnp.float32)
        return sT, vs

    def upd(sm, vs, carry):
        m_o, l_o, acc = carry
        m_n = jnp.maximum(m_o, jnp.max(sm, axis=0, keepdims=True))
        alpha = jnp.exp(m_o - m_n)
        e = jnp.exp(sm - m_n)
        l_n = l_o * alpha + jnp.sum(e, axis=0, keepdims=True)
        pv = jax.lax.dot_general(vs, e.astype(jnp.bfloat16),
                                 (((0,), (0,)), ((), ())),
                                 preferred_element_type=jnp.float32)
        return m_n, l_n, acc * alpha + pv

    init = (jnp.full((1, _R), _NEG, jnp.float32),
            jnp.zeros((1, _R), jnp.float32),
            jnp.zeros((_D, _R), jnp.float32))

    def body2(c, carry):
        sT, vs = qk(c)
        sm = sT * scale + w2_s[pl.ds(c * _CH, _CH), :]
        return upd(sm, vs, carry)

    car2 = jax.lax.fori_loop(0, qj, body2, init)

    jsub = jax.lax.broadcasted_iota(jnp.int32, (_CH, 1), 0)

    def body3(c, carry):
        sT, vs = qk(c)
        sm = sT * scale + jnp.where(_CH * c + jsub > tval - _WIN, 0.0, _NEG)
        return upd(sm, vs, carry)

    car3 = jax.lax.fori_loop(jnp.maximum(qj - 2, 0), qj, body3, init)

    sT, vs = qk(qj)
    ssc = sT * scale
    cadd2 = jnp.where(_CH * qj + jsub <= tval, 0.0, _NEG)
    car2 = upd(ssc + w2_s[pl.ds(qj * _CH, _CH), :] + cadd2, vs, car2)
    car3 = upd(ssc + cadd2, vs, car3)

    out_selT = car2[2] / jnp.maximum(car2[1], 1e-20)
    out_winT = car3[2] / jnp.maximum(car3[1], 1e-20)

    gT = jax.nn.sigmoid(jax.lax.dot_general(
        gw_ref[...], q2, (((0,), (1,)), ((), ())),
        preferred_element_type=jnp.float32))
    outT = (gT[0:1] * out_cmpT + gT[1:2] * out_selT + gT[2:3] * out_winT)
    out_ref[0, 0] = outT


def _half_band(w_half, lo):
    off = jnp.arange(_S)[None, :] - _STR * jnp.arange(_M)[:, None] - lo
    valid = (off >= 0) & (off < _STR)
    band = jnp.where(valid, w_half[jnp.clip(off, 0, _STR - 1)], 0.0)
    return jnp.pad(band, ((0, 1), (0, 0))).astype(jnp.bfloat16)


@functools.partial(jax.jit, static_argnames=("interpret",))
def _nsa(q, k, v, w_k, w_v, pe_k, pe_v, gate_w, interpret=False):
    qt = q[0].reshape(_S, _HKV, _REP, _D).transpose(1, 2, 0, 3)
    kt = k[0].transpose(1, 0, 2)
    vt = v[0].transpose(1, 0, 2)
    ktb = kt.astype(jnp.bfloat16)
    vtb = vt.astype(jnp.bfloat16)

    pea_k = jnp.tile(pe_k[:_STR], (_S // _STR, 1))
    peb_k = jnp.tile(pe_k[_STR:], (_S // _STR, 1))
    pea_v = jnp.tile(pe_v[:_STR], (_S // _STR, 1))
    peb_v = jnp.tile(pe_v[_STR:], (_S // _STR, 1))
    ka = (kt + pea_k[None]).astype(jnp.bfloat16)
    kb = (kt + peb_k[None]).astype(jnp.bfloat16)
    va = (vt + pea_v[None]).astype(jnp.bfloat16)
    vb = (vt + peb_v[None]).astype(jnp.bfloat16)

    b1k = _half_band(w_k[:_STR], 0)
    b2k = _half_band(w_k[_STR:], _STR)
    b1v = _half_band(w_v[:_STR], 0)
    b2v = _half_band(w_v[_STR:], _STR)

    ws = jnp.zeros((8, 128), jnp.float32)
    ws = ws.at[0, 0].set(jnp.maximum(jnp.sum(w_k), 1e-6))
    ws = ws.at[1, 0].set(jnp.maximum(jnp.sum(w_v), 1e-6))
    gw = jnp.pad(gate_w, ((0, 0), (0, 5))).astype(jnp.bfloat16)

    etok = (jnp.arange(_S)[:, None] // _BLK
            == jnp.arange(_NB)[None, :]).astype(jnp.bfloat16)
    rsum = (jnp.arange(_R)[:, None] % _G
            == jnp.arange(_G)[None, :]).astype(jnp.float32)
    em = (jnp.arange(128)[None, :] // 4
          == jnp.arange(_NB)[:, None]).astype(jnp.float32)

    out_t = pl.pallas_call(
        _nsa_kernel,
        grid=(_HKV, _NG),
        in_specs=[
            pl.BlockSpec((1, _REP, _G, _D), lambda h, qj: (h, 0, qj, 0)),
            pl.BlockSpec((1, _S, _D), lambda h, qj: (h, 0, 0)),
            pl.BlockSpec((1, _S, _D), lambda h, qj: (h, 0, 0)),
            pl.BlockSpec((1, _S, _D), lambda h, qj: (h, 0, 0)),
            pl.BlockSpec((1, _S, _D), lambda h, qj: (h, 0, 0)),
            pl.BlockSpec((1, _S, _D), lambda h, qj: (h, 0, 0)),
            pl.BlockSpec((1, _S, _D), lambda h, qj: (h, 0, 0)),
            pl.BlockSpec((128, _S), lambda h, qj: (0, 0)),
            pl.BlockSpec((128, _S), lambda h, qj: (0, 0)),
            pl.BlockSpec((128, _S), lambda h, qj: (0, 0)),
            pl.BlockSpec((128, _S), lambda h, qj: (0, 0)),
            pl.BlockSpec((8, 128), lambda h, qj: (0, 0)),
            pl.BlockSpec((_D, 8), lambda h, qj: (0, 0)),
            pl.BlockSpec((_S, _NB), lambda h, qj: (0, 0)),
            pl.BlockSpec((_R, _G), lambda h, qj: (0, 0)),
            pl.BlockSpec((_NB, 128), lambda h, qj: (0, 0)),
        ],
        out_specs=pl.BlockSpec((1, 1, _D, _R), lambda h, qj: (h, qj, 0, 0)),
        out_shape=jax.ShapeDtypeStruct((_HKV, _NG, _D, _R), jnp.float32),
        scratch_shapes=[
            pltpu.VMEM((128, _D), jnp.bfloat16),
            pltpu.VMEM((128, _D), jnp.bfloat16),
            pltpu.VMEM((_S, _R), jnp.float32),
        ],
        interpret=interpret,
    )(qt, ktb, vtb, ka, kb, va, vb, b1k, b2k, b1v, b2v, ws, gw,
      etok, rsum, em)

    out = (out_t.reshape(_HKV, _NG, _D, _REP, _G)
           .transpose(1, 4, 0, 3, 2).reshape(1, _S, _HQ, _D))
    return out


def kernel(q, k, v, w_k, w_v, pe_k, pe_v, gate_w):
    return _nsa(q, k, v, w_k, w_v, pe_k, pe_v, gate_w)


# speedup vs baseline: 5.2829x; 1.0017x over previous
import functools
import math

import jax
import jax.numpy as jnp
from jax.experimental import pallas as pl
from jax.experimental.pallas import tpu as pltpu

_S = 2048
_HQ = 16
_HKV = 4
_REP = _HQ // _HKV
_D = 128
_KER = 32
_STR = 16
_BLK = 64
_TOPN = 16
_WIN = 512
_M = (_S - _KER) // _STR + 1
_NB = _S // _BLK
_G = 256
_NG = _S // _G
_CH = 256
_R = _REP * _G
_NEG = -1e30
_HI = jax.lax.Precision.HIGHEST


def _nsa_kernel(qt_ref, ktb_ref, vtb_ref, ka_ref, kb_ref, va_ref, vb_ref,
                b1k_ref, b2k_ref, b1v_ref, b2v_ref, ws_ref, gw_ref,
                etok_ref, rsum_ref, em_ref, out_ref, cks, cvs):
    qj = pl.program_id(1)
    scale = 1.0 / math.sqrt(_D)

    @pl.when(qj == 0)
    def _compress_kv():
        wsk = ws_ref[0:1, 0:1]
        wsv = ws_ref[1:2, 0:1]
        ck = (jnp.dot(b1k_ref[...], ka_ref[0], preferred_element_type=jnp.float32)
              + jnp.dot(b2k_ref[...], kb_ref[0], preferred_element_type=jnp.float32))
        cv = (jnp.dot(b1v_ref[...], va_ref[0], preferred_element_type=jnp.float32)
              + jnp.dot(b2v_ref[...], vb_ref[0], preferred_element_type=jnp.float32))
        cks[...] = (ck / wsk).astype(jnp.bfloat16)
        cvs[...] = (cv / wsv).astype(jnp.bfloat16)

    q2 = qt_ref[0].reshape(_R, _D).astype(jnp.bfloat16)
    tl = jax.lax.broadcasted_iota(jnp.int32, (1, _R), 1) % _G
    tval = _G * qj + tl

    scT = jax.lax.dot_general(cks[...], q2, (((1,), (1,)), ((), ())),
                              preferred_element_type=jnp.float32)
    m_sub = jax.lax.broadcasted_iota(jnp.int32, (128, 1), 0)
    cadd = jnp.where((_STR * m_sub + _KER - 1 <= tval) & (m_sub < _M),
                     0.0, _NEG)
    scm = scT * scale + cadd
    cmx = jnp.max(scm, axis=0, keepdims=True)
    ce = jnp.exp(scm - cmx)
    cden = jnp.sum(ce, axis=0, keepdims=True)
    pcT = ce / jnp.maximum(cden, 1e-20)
    out_cmpT = jax.lax.dot_general(cvs[...], pcT.astype(jnp.bfloat16),
                                   (((0,), (0,)), ((), ())),
                                   preferred_element_type=jnp.float32)
    out_cmpT = out_cmpT * jnp.where(tval >= _KER - 1, 1.0, 0.0)

    pgT = jax.lax.dot_general(pcT, rsum_ref[...], (((1,), (0,)), ((), ())),
                              precision=_HI,
                              preferred_element_type=jnp.float32)
    selT = jax.lax.dot_general(em_ref[...], pgT, (((1,), (0,)), ((), ())),
                               precision=_HI,
                               preferred_element_type=jnp.float32)
    nnS = jax.lax.broadcasted_iota(jnp.int32, (_NB, 1), 0)
    cur = 4 * qj + jax.lax.broadcasted_iota(jnp.int32, (1, _G), 1) // _BLK
    selT = selT + jnp.where((nnS == cur) | (nnS == 0), 1e9, 0.0)
    selw = selT
    picked = jnp.zeros((_NB, _G), jnp.bool_)
    for _ in range(_TOPN):
        mx = jnp.max(selw, axis=0, keepdims=True)
        idx = jnp.where(selw == mx, nnS, _NB)
        fidx = jnp.min(idx, axis=0, keepdims=True)
        pick = nnS == fidx
        picked = picked | pick
        selw = jnp.where(pick, -jnp.inf, selw)
    blk_add = jnp.where(picked, 0.0, _NEG).astype(jnp.bfloat16)
    blk_add4 = jnp.concatenate([blk_add] * _REP, axis=1)

    def w2(c):
        return jax.lax.dot_general(etok_ref[pl.ds(c * _CH, _CH), :], blk_add4,
                                   (((1,), (0,)), ((), ())),
                                   preferred_element_type=jnp.float32)

    def qk(c):
        ks = ktb_ref[0, pl.ds(c * _CH, _CH), :]
        vs = vtb_ref[0, pl.ds(c * _CH, _CH), :]
        sT = jax.lax.dot_general(ks, q2, (((1,), (1,)), ((), ())),
                                 preferred_element_type=jnp.float32)
        return sT, vs

    def upd(sm, vs, carry):
        m_o, l_o, acc = carry
        m_n = jnp.maximum(m_o, jnp.max(sm, axis=0, keepdims=True))
        alpha = jnp.exp(m_o - m_n)
        e = jnp.exp(sm - m_n)
        l_n = l_o * alpha + jnp.sum(e, axis=0, keepdims=True)
        pv = jax.lax.dot_general(vs, e.astype(jnp.bfloat16),
                                 (((0,), (0,)), ((), ())),
                                 preferred_element_type=jnp.float32)
        return m_n, l_n, acc * alpha + pv

    init = (jnp.full((1, _R), _NEG, jnp.float32),
            jnp.zeros((1, _R), jnp.float32),
            jnp.zeros((_D, _R), jnp.float32))

    def body2(c, carry):
        sT, vs = qk(c)
        sm = sT * scale + w2(c)
        return upd(sm, vs, carry)

    car2 = jax.lax.fori_loop(0, qj, body2, init)

    jsub = jax.lax.broadcasted_iota(jnp.int32, (_CH, 1), 0)

    def body3(c, carry):
        sT, vs = qk(c)
        sm = sT * scale + jnp.where(_CH * c + jsub > tval - _WIN, 0.0, _NEG)
        return upd(sm, vs, carry)

    car3 = jax.lax.fori_loop(jnp.maximum(qj - 2, 0), qj, body3, init)

    sT, vs = qk(qj)
    ssc = sT * scale
    cadd2 = jnp.where(_CH * qj + jsub <= tval, 0.0, _NEG)
    car2 = upd(ssc + w2(qj) + cadd2, vs, car2)
    car3 = upd(ssc + cadd2, vs, car3)

    out_selT = car2[2] / jnp.maximum(car2[1], 1e-20)
    out_winT = car3[2] / jnp.maximum(car3[1], 1e-20)

    gT = jax.nn.sigmoid(jax.lax.dot_general(
        gw_ref[...], q2, (((0,), (1,)), ((), ())),
        preferred_element_type=jnp.float32))
    outT = (gT[0:1] * out_cmpT + gT[1:2] * out_selT + gT[2:3] * out_winT)
    out_ref[0, 0] = outT


def _half_band(w_half, lo):
    off = jnp.arange(_S)[None, :] - _STR * jnp.arange(_M)[:, None] - lo
    valid = (off >= 0) & (off < _STR)
    band = jnp.where(valid, w_half[jnp.clip(off, 0, _STR - 1)], 0.0)
    return jnp.pad(band, ((0, 1), (0, 0))).astype(jnp.bfloat16)


@functools.partial(jax.jit, static_argnames=("interpret",))
def _nsa(q, k, v, w_k, w_v, pe_k, pe_v, gate_w, interpret=False):
    qt = q[0].reshape(_S, _HKV, _REP, _D).transpose(1, 2, 0, 3)
    kt = k[0].transpose(1, 0, 2)
    vt = v[0].transpose(1, 0, 2)
    ktb = kt.astype(jnp.bfloat16)
    vtb = vt.astype(jnp.bfloat16)

    pea_k = jnp.tile(pe_k[:_STR], (_S // _STR, 1))
    peb_k = jnp.tile(pe_k[_STR:], (_S // _STR, 1))
    pea_v = jnp.tile(pe_v[:_STR], (_S // _STR, 1))
    peb_v = jnp.tile(pe_v[_STR:], (_S // _STR, 1))
    ka = (kt + pea_k[None]).astype(jnp.bfloat16)
    kb = (kt + peb_k[None]).astype(jnp.bfloat16)
    va = (vt + pea_v[None]).astype(jnp.bfloat16)
    vb = (vt + peb_v[None]).astype(jnp.bfloat16)

    b1k = _half_band(w_k[:_STR], 0)
    b2k = _half_band(w_k[_STR:], _STR)
    b1v = _half_band(w_v[:_STR], 0)
    b2v = _half_band(w_v[_STR:], _STR)

    ws = jnp.zeros((8, 128), jnp.float32)
    ws = ws.at[0, 0].set(jnp.maximum(jnp.sum(w_k), 1e-6))
    ws = ws.at[1, 0].set(jnp.maximum(jnp.sum(w_v), 1e-6))
    gw = jnp.pad(gate_w, ((0, 0), (0, 5))).astype(jnp.bfloat16)

    etok = (jnp.arange(_S)[:, None] // _BLK
            == jnp.arange(_NB)[None, :]).astype(jnp.bfloat16)
    rsum = (jnp.arange(_R)[:, None] % _G
            == jnp.arange(_G)[None, :]).astype(jnp.float32)
    em = (jnp.arange(128)[None, :] // 4
          == jnp.arange(_NB)[:, None]).astype(jnp.float32)

    out_t = pl.pallas_call(
        _nsa_kernel,
        grid=(_HKV, _NG),
        in_specs=[
            pl.BlockSpec((1, _REP, _G, _D), lambda h, qj: (h, 0, qj, 0)),
            pl.BlockSpec((1, _S, _D), lambda h, qj: (h, 0, 0)),
            pl.BlockSpec((1, _S, _D), lambda h, qj: (h, 0, 0)),
            pl.BlockSpec((1, _S, _D), lambda h, qj: (h, 0, 0)),
            pl.BlockSpec((1, _S, _D), lambda h, qj: (h, 0, 0)),
            pl.BlockSpec((1, _S, _D), lambda h, qj: (h, 0, 0)),
            pl.BlockSpec((1, _S, _D), lambda h, qj: (h, 0, 0)),
            pl.BlockSpec((128, _S), lambda h, qj: (0, 0)),
            pl.BlockSpec((128, _S), lambda h, qj: (0, 0)),
            pl.BlockSpec((128, _S), lambda h, qj: (0, 0)),
            pl.BlockSpec((128, _S), lambda h, qj: (0, 0)),
            pl.BlockSpec((8, 128), lambda h, qj: (0, 0)),
            pl.BlockSpec((_D, 8), lambda h, qj: (0, 0)),
            pl.BlockSpec((_S, _NB), lambda h, qj: (0, 0)),
            pl.BlockSpec((_R, _G), lambda h, qj: (0, 0)),
            pl.BlockSpec((_NB, 128), lambda h, qj: (0, 0)),
        ],
        out_specs=pl.BlockSpec((1, 1, _D, _R), lambda h, qj: (h, qj, 0, 0)),
        out_shape=jax.ShapeDtypeStruct((_HKV, _NG, _D, _R), jnp.float32),
        scratch_shapes=[
            pltpu.VMEM((128, _D), jnp.bfloat16),
            pltpu.VMEM((128, _D), jnp.bfloat16),
        ],
        interpret=interpret,
    )(qt, ktb, vtb, ka, kb, va, vb, b1k, b2k, b1v, b2v, ws, gw,
      etok, rsum, em)

    out = (out_t.reshape(_HKV, _NG, _D, _REP, _G)
           .transpose(1, 4, 0, 3, 2).reshape(1, _S, _HQ, _D))
    return out


def kernel(q, k, v, w_k, w_v, pe_k, pe_v, gate_w):
    return _nsa(q, k, v, w_k, w_v, pe_k, pe_v, gate_w)
